# baseline (speedup 1.0000x reference)
import jax
import jax.numpy as jnp
from jax.experimental import pallas as pl
from jax.experimental.pallas import tpu as pltpu

SUB, LANE = 8, 4096


def _body(sched_ref, inds_ref, x_ref, out_ref):
    i = pl.program_id(0)

    @pl.when(i == 0)
    def _init():
        out_ref[...] = jnp.ones((1, 1), jnp.float32)

    row = sched_ref[i]
    m = inds_ref[row] <= 0
    v = x_ref[...]
    bad = m & jnp.logical_not(jnp.all(v == v))
    out_ref[...] = out_ref[...] * jnp.where(bad, 0.0, 1.0).astype(jnp.float32)


def kernel(x, inds):
    r, c = x.shape
    inds2 = jnp.asarray(inds, dtype=jnp.int32)
    x3 = x.reshape(r, SUB, c // SUB)
    iota = jnp.arange(r, dtype=jnp.int32)
    sched = jnp.maximum(jax.lax.cummax(jnp.where(inds2 <= 0, iota, -1)), 0)
    out = pl.pallas_call(
        _body,
        grid_spec=pltpu.PrefetchScalarGridSpec(
            num_scalar_prefetch=1,
            grid=(r,),
            in_specs=[
                pl.BlockSpec(memory_space=pltpu.SMEM),
                pl.BlockSpec((1, SUB, LANE), lambda i, sched: (sched[i], 0, 0)),
            ],
            out_specs=pl.BlockSpec((1, 1), lambda i, sched: (0, 0)),
        ),
        out_shape=jax.ShapeDtypeStruct((1, 1), jnp.float32),
    )(sched, inds2, x3)
    return out.reshape(1)


# speedup vs baseline: 10.0622x; 10.0622x over previous
import jax
import jax.numpy as jnp
from jax.experimental import pallas as pl


def _body(inds_ref, x_ref, out_ref):
    i = pl.program_id(0)

    @pl.when(i == 0)
    def _init():
        out_ref[...] = jnp.ones((1, 1), jnp.float32)

    xb = x_ref[...]
    m = inds_ref[...] <= 0
    v = jnp.where(m, xb, jnp.float32(0.0))
    ok = jnp.all(v == v)
    out_ref[...] = out_ref[...] * jnp.where(ok, 1.0, 0.0).astype(jnp.float32)


def kernel(x, inds):
    r, c = x.shape
    inds2 = jnp.asarray(inds, dtype=jnp.int32).reshape(r, 1)
    blk_c = 16384
    grid = (c // blk_c,)
    out = pl.pallas_call(
        _body,
        grid=grid,
        in_specs=[
            pl.BlockSpec((r, 1), lambda i: (0, 0)),
            pl.BlockSpec((r, blk_c), lambda i: (0, i)),
        ],
        out_specs=pl.BlockSpec((1, 1), lambda i: (0, 0)),
        out_shape=jax.ShapeDtypeStruct((1, 1), jnp.float32),
    )(inds2, x)
    return out.reshape(1)
